# Initial kernel scaffold; baseline (speedup 1.0000x reference)
import jax
import jax.numpy as jnp
from jax.experimental import pallas as pl


def kernel(x, tables):
    raise NotImplementedError("write your pallas kernel here")


# speedup vs baseline: 1.0285x; 1.0285x over previous
import functools

import jax
import jax.numpy as jnp
from jax import lax
from jax.experimental import pallas as pl
from jax.experimental.pallas import tpu as pltpu
from jax.experimental.pallas import tpu_sc as plsc

NUM_FIELDS = 26
VOCAB = 100000
DIM = 8
BATCH = 16384

NC, NS = 2, 16
NW = NC * NS
B_FLAT = BATCH * NUM_FIELDS
B_PER_W = B_FLAT // NW
CHUNK = 128
N_CHUNKS = B_PER_W // CHUNK
GROUP = 8
N_GROUPS = N_CHUNKS // GROUP
X_ROWS_PER_W = B_PER_W // CHUNK


def _body(x_hbm, table_hbm, out_hbm, idx_v, rows_v, sem):
    cid = lax.axis_index("c")
    sid = lax.axis_index("s")
    wid = sid * NC + cid

    pltpu.sync_copy(x_hbm.at[pl.ds(wid * X_ROWS_PER_W, X_ROWS_PER_W)], idx_v)

    lane = lax.iota(jnp.int32, 16)

    def off_body(j, carry):
        for r in range(CHUNK // 16):
            pos = j * CHUNK + r * 16 + lane
            field = lax.rem(pos, NUM_FIELDS)
            sl = pl.ds(r * 16, 16)
            idx_v[j, sl] = idx_v[j, sl] + field * VOCAB
        return carry

    lax.fori_loop(0, N_CHUNKS, off_body, 0, unroll=False)

    def gather_body(g, carry):
        descs = []
        for b in range(GROUP):
            j = g * GROUP + b
            descs.append(
                pltpu.async_copy(
                    table_hbm.at[idx_v.at[j]],
                    rows_v.at[pl.ds(j * CHUNK, CHUNK)],
                    sem,
                )
            )
        for d in descs:
            d.wait()
        return carry

    lax.fori_loop(0, N_GROUPS, gather_body, 0, unroll=False)

    pltpu.sync_copy(rows_v, out_hbm.at[pl.ds(wid * B_PER_W, B_PER_W)])


@jax.jit
def _sc_gather(x2d, table_flat):
    mesh = plsc.VectorSubcoreMesh(
        core_axis_name="c", subcore_axis_name="s", num_cores=NC, num_subcores=NS
    )
    return pl.kernel(
        _body,
        out_type=jax.ShapeDtypeStruct((B_FLAT, DIM), jnp.float32),
        mesh=mesh,
        scratch_types=[
            pltpu.VMEM((X_ROWS_PER_W, CHUNK), jnp.int32),
            pltpu.VMEM((B_PER_W, DIM), jnp.float32),
            pltpu.SemaphoreType.DMA,
        ],
        compiler_params=pltpu.CompilerParams(use_tc_tiling_on_sc=False),
    )(x2d, table_flat)


def kernel(x, tables):
    x2d = x.astype(jnp.int32).reshape(B_FLAT // CHUNK, CHUNK)
    table_flat = tables.reshape(NUM_FIELDS * VOCAB, DIM)
    out = _sc_gather(x2d, table_flat)
    return out.reshape(BATCH, NUM_FIELDS * DIM)
